# baseline (speedup 1.0000x reference)
import functools

import jax
import jax.numpy as jnp
from jax import lax
from jax.experimental import pallas as pl
from jax.experimental.pallas import tpu as pltpu
from jax.experimental.pallas import tpu_sc as plsc

_LANES = 16


def _make_mf_kernel(B, K, num_cores, num_subcores):
    NW = num_cores * num_subcores
    bpw = B // NW
    n_groups = bpw // _LANES

    mesh = plsc.VectorSubcoreMesh(core_axis_name="c", subcore_axis_name="s")

    @functools.partial(
        pl.kernel,
        out_type=jax.ShapeDtypeStruct((B,), jnp.float32),
        mesh=mesh,
        scratch_types=[
            pltpu.VMEM((bpw,), jnp.int32),
            pltpu.VMEM((bpw,), jnp.int32),
            pltpu.VMEM((bpw, K), jnp.float32),
            pltpu.VMEM((bpw, K), jnp.float32),
            pltpu.VMEM((bpw,), jnp.float32),
            pltpu.SemaphoreType.DMA,
        ],
        compiler_params=pltpu.CompilerParams(
            needs_layout_passes=False, use_tc_tiling_on_sc=False
        ),
    )
    def mf_kernel(uidx_hbm, vidx_hbm, w_hbm, h_hbm, out_hbm,
                  uidx_v, vidx_v, urows_v, vrows_v, out_v, sem):
        wid = lax.axis_index("s") * num_cores + lax.axis_index("c")
        base = wid * bpw

        pltpu.sync_copy(uidx_hbm.at[pl.ds(base, bpw)], uidx_v)
        pltpu.sync_copy(vidx_hbm.at[pl.ds(base, bpw)], vidx_v)

        cu = pltpu.async_copy(w_hbm.at[uidx_v], urows_v, sem)
        cv = pltpu.async_copy(h_hbm.at[vidx_v], vrows_v, sem)
        cu.wait()
        cv.wait()

        lanes = lax.iota(jnp.int32, _LANES)

        def body(g, carry):
            rows = g * _LANES + lanes
            acc = jnp.zeros((_LANES,), jnp.float32)
            for c in range(K):
                col = jnp.full((_LANES,), c, jnp.int32)
                uc = plsc.load_gather(urows_v, [rows, col])
                vc = plsc.load_gather(vrows_v, [rows, col])
                acc = acc + uc * vc
            sig = 1.0 / (1.0 + jnp.exp(-acc))
            out_v[pl.ds(g * _LANES, _LANES)] = sig
            return carry

        lax.fori_loop(0, n_groups, body, 0)

        pltpu.sync_copy(out_v, out_hbm.at[pl.ds(base, bpw)])

    return mf_kernel


def kernel(x, W, H):
    B = x.shape[0]
    K = W.shape[1]
    info = plsc.get_sparse_core_info()
    user_idx = x[:, 0]
    item_idx = x[:, 1]
    mf = _make_mf_kernel(B, K, info.num_cores, info.num_subcores)
    return mf(user_idx, item_idx, W, H)
